# Initial kernel scaffold; baseline (speedup 1.0000x reference)
import jax
import jax.numpy as jnp
from jax.experimental import pallas as pl


def kernel(h, c, edge_index, U_iou_W, U_f_W, U_f_b, b_iou):
    raise NotImplementedError("write your pallas kernel here")


# speedup vs baseline: 5.9133x; 5.9133x over previous
import functools

import jax
import jax.numpy as jnp
from jax import lax
from jax.experimental import pallas as pl
from jax.experimental.pallas import tpu as pltpu
from jax.experimental.pallas import tpu_sc as plsc

N = 10000
E = 320000
H = 128

NT = 16
K = 128
EPT_RAW = E // NT
NCHUNK = (EPT_RAW + K - 1) // K
EPT = NCHUNK * K
PAD = EPT - EPT_RAW
ACC_ROWS = 10240
ROWS_PER_TILE = ACC_ROWS // NT

_mesh = plsc.VectorSubcoreMesh(core_axis_name="c", subcore_axis_name="s")


@functools.partial(
    pl.kernel,
    out_type=[
        jax.ShapeDtypeStruct((ACC_ROWS, H), jnp.float32),
        jax.ShapeDtypeStruct((ACC_ROWS, H), jnp.float32),
    ],
    mesh=_mesh,
    scratch_types=[
        pltpu.VMEM((K,), jnp.int32),
        pltpu.VMEM((K,), jnp.int32),
        pltpu.VMEM((K, H), jnp.float32),
        pltpu.VMEM_SHARED((ACC_ROWS, H), jnp.float32),
        pltpu.SemaphoreType.DMA,
    ],
)
def _segment_sums(h_hbm, c_hbm, src_hbm, dst_hbm, hsum_hbm, csum_hbm,
                  src_v, dst_v, rows_v, acc_sh, sem):
    ci = lax.axis_index("c")
    si = lax.axis_index("s")

    zeros16 = jnp.zeros((16,), jnp.float32)
    def _zrow(r, carry):
        def _zcol(j, carry2):
            rows_v[r, pl.ds(j * 16, 16)] = zeros16
            return carry2
        return lax.fori_loop(0, H // 16, _zcol, carry)
    lax.fori_loop(0, K, _zrow, 0)
    def _zcopy(r, carry):
        pltpu.sync_copy(rows_v, acc_sh.at[pl.ds(si * ROWS_PER_TILE + r * K, K)])
        return carry
    lax.fori_loop(0, ROWS_PER_TILE // K, _zcopy, 0)
    plsc.subcore_barrier()

    def _run(table_hbm):
        def body(i, carry):
            base = i * K
            pltpu.sync_copy(src_hbm.at[si, pl.ds(base, K)], src_v)
            pltpu.sync_copy(dst_hbm.at[si, pl.ds(base, K)], dst_v)
            pltpu.async_copy(table_hbm.at[src_v], rows_v, sem).wait()
            pltpu.sync_copy(rows_v, acc_sh.at[dst_v], add=True)
            return carry
        lax.fori_loop(0, NCHUNK, body, 0)

    @pl.when(ci == 0)
    def _():
        _run(h_hbm)

    @pl.when(ci == 1)
    def _():
        _run(c_hbm)

    plsc.subcore_barrier()

    base = si * ROWS_PER_TILE

    @pl.when(ci == 0)
    def _():
        pltpu.sync_copy(acc_sh.at[pl.ds(base, ROWS_PER_TILE)],
                        hsum_hbm.at[pl.ds(base, ROWS_PER_TILE)])

    @pl.when(ci == 1)
    def _():
        pltpu.sync_copy(acc_sh.at[pl.ds(base, ROWS_PER_TILE)],
                        csum_hbm.at[pl.ds(base, ROWS_PER_TILE)])


BLK = 400


def _gates_body(hs_ref, cs_ref, ufw_ref, ufb_ref, uiou_ref, biou_ref,
                hnew_ref, cnew_ref):
    hs = hs_ref[...]
    dn = (((1,), (1,)), ((), ()))
    f = jax.nn.sigmoid(
        lax.dot_general(hs, ufw_ref[...], dn,
                        preferred_element_type=jnp.float32) + ufb_ref[...])
    c_agg = f * cs_ref[...]
    iou = lax.dot_general(hs, uiou_ref[...], dn,
                          preferred_element_type=jnp.float32) + biou_ref[...]
    i_g = jax.nn.sigmoid(iou[:, :H])
    o_g = jax.nn.sigmoid(iou[:, H:2 * H])
    u_g = jnp.tanh(iou[:, 2 * H:])
    c_new = i_g * u_g + c_agg
    cnew_ref[...] = c_new
    hnew_ref[...] = o_g * jnp.tanh(c_new)


_gates = pl.pallas_call(
    _gates_body,
    grid=(N // BLK,),
    in_specs=[
        pl.BlockSpec((BLK, H), lambda i: (i, 0)),
        pl.BlockSpec((BLK, H), lambda i: (i, 0)),
        pl.BlockSpec((H, H), lambda i: (0, 0)),
        pl.BlockSpec((1, H), lambda i: (0, 0)),
        pl.BlockSpec((3 * H, H), lambda i: (0, 0)),
        pl.BlockSpec((1, 3 * H), lambda i: (0, 0)),
    ],
    out_specs=[
        pl.BlockSpec((BLK, H), lambda i: (i, 0)),
        pl.BlockSpec((BLK, H), lambda i: (i, 0)),
    ],
    out_shape=[
        jax.ShapeDtypeStruct((N, H), jnp.float32),
        jax.ShapeDtypeStruct((N, H), jnp.float32),
    ],
)


@jax.jit
def kernel(h, c, edge_index, U_iou_W, U_f_W, U_f_b, b_iou):
    src = edge_index[0]
    dst = edge_index[1]
    src_p = jnp.concatenate(
        [src.reshape(NT, EPT_RAW),
         jnp.zeros((NT, PAD), jnp.int32)], axis=1)
    dst_p = jnp.concatenate(
        [dst.reshape(NT, EPT_RAW),
         jnp.full((NT, PAD), N, jnp.int32)], axis=1)
    h_sum, c_sum = _segment_sums(h, c, src_p, dst_p)
    h_new, c_new = _gates(h_sum, c_sum, U_f_W, U_f_b.reshape(1, H),
                          U_iou_W, b_iou)
    return (h_new, c_new)


# speedup vs baseline: 6.1338x; 1.0373x over previous
import functools

import jax
import jax.numpy as jnp
from jax import lax
from jax.experimental import pallas as pl
from jax.experimental.pallas import tpu as pltpu
from jax.experimental.pallas import tpu_sc as plsc

N = 10000
E = 320000
H = 128

NT = 16
K = 128
IB = 32
NBLK = 5
EPT_RAW = E // NT
EPT = NBLK * IB * K
PAD = EPT - EPT_RAW
ACC_ROWS = 10240
ROWS_PER_TILE = ACC_ROWS // NT

_mesh = plsc.VectorSubcoreMesh(core_axis_name="c", subcore_axis_name="s")


@functools.partial(
    pl.kernel,
    out_type=[
        jax.ShapeDtypeStruct((ACC_ROWS, H), jnp.float32),
        jax.ShapeDtypeStruct((ACC_ROWS, H), jnp.float32),
    ],
    mesh=_mesh,
    scratch_types=[
        pltpu.VMEM((IB, K), jnp.int32),
        pltpu.VMEM((IB, K), jnp.int32),
        pltpu.VMEM((K, H), jnp.float32),
        pltpu.VMEM((K, H), jnp.float32),
        pltpu.VMEM_SHARED((ACC_ROWS, H), jnp.float32),
        pltpu.SemaphoreType.DMA,
        pltpu.SemaphoreType.DMA,
    ],
)
def _segment_sums(h_hbm, c_hbm, src_hbm, dst_hbm, hsum_hbm, csum_hbm,
                  src_v, dst_v, buf0, buf1, acc_sh, sem0, sem1):
    ci = lax.axis_index("c")
    si = lax.axis_index("s")

    zeros16 = jnp.zeros((16,), jnp.float32)
    def _zrow(r, carry):
        def _zcol(j, carry2):
            buf0[r, pl.ds(j * 16, 16)] = zeros16
            return carry2
        return lax.fori_loop(0, H // 16, _zcol, carry)
    lax.fori_loop(0, K, _zrow, 0)
    def _zcopy(r, carry):
        pltpu.sync_copy(buf0, acc_sh.at[pl.ds(si * ROWS_PER_TILE + r * K, K)])
        return carry
    lax.fori_loop(0, ROWS_PER_TILE // K, _zcopy, 0)
    plsc.subcore_barrier()

    def _run(table_hbm):
        def blk(b, carry):
            pltpu.sync_copy(src_hbm.at[si, b], src_v)
            pltpu.sync_copy(dst_hbm.at[si, b], dst_v)
            pltpu.async_copy(table_hbm.at[src_v.at[0]], buf0, sem0)
            def inner(jh, c2):
                j0 = 2 * jh
                j1 = j0 + 1
                pltpu.async_copy(table_hbm.at[src_v.at[j1]], buf1, sem1)
                pltpu.make_async_copy(table_hbm.at[src_v.at[j0]],
                                      buf0, sem0).wait()
                pltpu.sync_copy(buf0, acc_sh.at[dst_v.at[j0]], add=True)
                @pl.when(jh < IB // 2 - 1)
                def _():
                    pltpu.async_copy(table_hbm.at[src_v.at[j0 + 2]],
                                     buf0, sem0)
                pltpu.make_async_copy(table_hbm.at[src_v.at[j1]],
                                      buf1, sem1).wait()
                pltpu.sync_copy(buf1, acc_sh.at[dst_v.at[j1]], add=True)
                return c2
            lax.fori_loop(0, IB // 2, inner, 0)
            return carry
        lax.fori_loop(0, NBLK, blk, 0)

    @pl.when(ci == 0)
    def _():
        _run(h_hbm)

    @pl.when(ci == 1)
    def _():
        _run(c_hbm)

    plsc.subcore_barrier()

    base = si * ROWS_PER_TILE

    @pl.when(ci == 0)
    def _():
        pltpu.sync_copy(acc_sh.at[pl.ds(base, ROWS_PER_TILE)],
                        hsum_hbm.at[pl.ds(base, ROWS_PER_TILE)])

    @pl.when(ci == 1)
    def _():
        pltpu.sync_copy(acc_sh.at[pl.ds(base, ROWS_PER_TILE)],
                        csum_hbm.at[pl.ds(base, ROWS_PER_TILE)])


BLK = 400


def _gates_body(hs_ref, cs_ref, ufw_ref, ufb_ref, uiou_ref, biou_ref,
                hnew_ref, cnew_ref):
    hs = hs_ref[...]
    dn = (((1,), (1,)), ((), ()))
    f = jax.nn.sigmoid(
        lax.dot_general(hs, ufw_ref[...], dn,
                        preferred_element_type=jnp.float32) + ufb_ref[...])
    c_agg = f * cs_ref[...]
    iou = lax.dot_general(hs, uiou_ref[...], dn,
                          preferred_element_type=jnp.float32) + biou_ref[...]
    i_g = jax.nn.sigmoid(iou[:, :H])
    o_g = jax.nn.sigmoid(iou[:, H:2 * H])
    u_g = jnp.tanh(iou[:, 2 * H:])
    c_new = i_g * u_g + c_agg
    cnew_ref[...] = c_new
    hnew_ref[...] = o_g * jnp.tanh(c_new)


_gates = pl.pallas_call(
    _gates_body,
    grid=(N // BLK,),
    in_specs=[
        pl.BlockSpec((BLK, H), lambda i: (i, 0)),
        pl.BlockSpec((BLK, H), lambda i: (i, 0)),
        pl.BlockSpec((H, H), lambda i: (0, 0)),
        pl.BlockSpec((1, H), lambda i: (0, 0)),
        pl.BlockSpec((3 * H, H), lambda i: (0, 0)),
        pl.BlockSpec((1, 3 * H), lambda i: (0, 0)),
    ],
    out_specs=[
        pl.BlockSpec((BLK, H), lambda i: (i, 0)),
        pl.BlockSpec((BLK, H), lambda i: (i, 0)),
    ],
    out_shape=[
        jax.ShapeDtypeStruct((N, H), jnp.float32),
        jax.ShapeDtypeStruct((N, H), jnp.float32),
    ],
)


@jax.jit
def kernel(h, c, edge_index, U_iou_W, U_f_W, U_f_b, b_iou):
    src = edge_index[0]
    dst = edge_index[1]
    src_p = jnp.concatenate(
        [src.reshape(NT, EPT_RAW),
         jnp.zeros((NT, PAD), jnp.int32)], axis=1).reshape(NT, NBLK, IB, K)
    dst_p = jnp.concatenate(
        [dst.reshape(NT, EPT_RAW),
         jnp.full((NT, PAD), N, jnp.int32)], axis=1).reshape(NT, NBLK, IB, K)
    h_sum, c_sum = _segment_sums(h, c, src_p, dst_p)
    h_new, c_new = _gates(h_sum, c_sum, U_f_W, U_f_b.reshape(1, H),
                          U_iou_W, b_iou)
    return (h_new, c_new)


# speedup vs baseline: 6.1772x; 1.0071x over previous
import functools

import jax
import jax.numpy as jnp
from jax import lax
from jax.experimental import pallas as pl
from jax.experimental.pallas import tpu as pltpu
from jax.experimental.pallas import tpu_sc as plsc

N = 10000
E = 320000
H = 128

NT = 16
K = 64
IB = 64
NBLK = 5
EPT_RAW = E // NT
EPT = NBLK * IB * K
PAD = EPT - EPT_RAW
ACC_ROWS = 10240
ROWS_PER_TILE = ACC_ROWS // NT

_mesh = plsc.VectorSubcoreMesh(core_axis_name="c", subcore_axis_name="s")


@functools.partial(
    pl.kernel,
    out_type=[
        jax.ShapeDtypeStruct((ACC_ROWS, H), jnp.float32),
        jax.ShapeDtypeStruct((ACC_ROWS, H), jnp.float32),
    ],
    mesh=_mesh,
    scratch_types=[
        pltpu.VMEM((IB, K), jnp.int32),
        pltpu.VMEM((IB, K), jnp.int32),
        pltpu.VMEM((4, K, H), jnp.float32),
        pltpu.VMEM_SHARED((ACC_ROWS, H), jnp.float32),
        [pltpu.SemaphoreType.DMA] * 4,
        [pltpu.SemaphoreType.DMA] * 4,
    ],
)
def _segment_sums(h_hbm, c_hbm, src_hbm, dst_hbm, hsum_hbm, csum_hbm,
                  src_v, dst_v, bufs, acc_sh, gsems, ssems):
    ci = lax.axis_index("c")
    si = lax.axis_index("s")
    buf = [bufs.at[m] for m in range(4)]

    zeros16 = jnp.zeros((16,), jnp.float32)
    def _zrow(r, carry):
        def _zcol(j, carry2):
            bufs[0, r, pl.ds(j * 16, 16)] = zeros16
            return carry2
        return lax.fori_loop(0, H // 16, _zcol, carry)
    lax.fori_loop(0, K, _zrow, 0)
    def _zcopy(r, carry):
        pltpu.sync_copy(buf[0], acc_sh.at[pl.ds(si * ROWS_PER_TILE + r * K, K)])
        return carry
    lax.fori_loop(0, ROWS_PER_TILE // K, _zcopy, 0)
    plsc.subcore_barrier()

    def _run(table_hbm):
        def gfire(j, m):
            pltpu.async_copy(table_hbm.at[src_v.at[j]], buf[m], gsems[m])
        def gwait(m):
            pltpu.make_async_copy(table_hbm.at[src_v.at[0]],
                                  buf[m], gsems[m]).wait()
        def sfire(j, m):
            pltpu.async_copy(buf[m], acc_sh.at[dst_v.at[j]], ssems[m],
                             add=True)
        def swait(m):
            pltpu.make_async_copy(buf[m], acc_sh.at[dst_v.at[0]],
                                  ssems[m]).wait()

        def blk(b, carry):
            pltpu.sync_copy(src_hbm.at[si, b], src_v)
            pltpu.sync_copy(dst_hbm.at[si, b], dst_v)
            gfire(0, 0)
            gfire(1, 1)
            gfire(2, 2)
            gwait(0)
            sfire(0, 0)
            gfire(3, 3)
            gwait(1)
            sfire(1, 1)
            def grp(jg, c2):
                jb = 4 * jg + 2
                for m in range(4):
                    j = jb + m
                    swait(m)
                    gfire(j + 2, m)
                    bs = (2 + m) % 4
                    gwait(bs)
                    sfire(j, bs)
                return c2
            lax.fori_loop(0, (IB - 4) // 4, grp, 0)
            gwait(2)
            sfire(IB - 2, 2)
            gwait(3)
            sfire(IB - 1, 3)
            swait(0)
            swait(1)
            swait(2)
            swait(3)
            return carry
        lax.fori_loop(0, NBLK, blk, 0)

    @pl.when(ci == 0)
    def _():
        _run(h_hbm)

    @pl.when(ci == 1)
    def _():
        _run(c_hbm)

    plsc.subcore_barrier()

    base = si * ROWS_PER_TILE

    @pl.when(ci == 0)
    def _():
        pltpu.sync_copy(acc_sh.at[pl.ds(base, ROWS_PER_TILE)],
                        hsum_hbm.at[pl.ds(base, ROWS_PER_TILE)])

    @pl.when(ci == 1)
    def _():
        pltpu.sync_copy(acc_sh.at[pl.ds(base, ROWS_PER_TILE)],
                        csum_hbm.at[pl.ds(base, ROWS_PER_TILE)])


BLK = 400


def _gates_body(hs_ref, cs_ref, ufw_ref, ufb_ref, uiou_ref, biou_ref,
                hnew_ref, cnew_ref):
    hs = hs_ref[...]
    dn = (((1,), (1,)), ((), ()))
    f = jax.nn.sigmoid(
        lax.dot_general(hs, ufw_ref[...], dn,
                        preferred_element_type=jnp.float32) + ufb_ref[...])
    c_agg = f * cs_ref[...]
    iou = lax.dot_general(hs, uiou_ref[...], dn,
                          preferred_element_type=jnp.float32) + biou_ref[...]
    i_g = jax.nn.sigmoid(iou[:, :H])
    o_g = jax.nn.sigmoid(iou[:, H:2 * H])
    u_g = jnp.tanh(iou[:, 2 * H:])
    c_new = i_g * u_g + c_agg
    cnew_ref[...] = c_new
    hnew_ref[...] = o_g * jnp.tanh(c_new)


_gates = pl.pallas_call(
    _gates_body,
    grid=(N // BLK,),
    in_specs=[
        pl.BlockSpec((BLK, H), lambda i: (i, 0)),
        pl.BlockSpec((BLK, H), lambda i: (i, 0)),
        pl.BlockSpec((H, H), lambda i: (0, 0)),
        pl.BlockSpec((1, H), lambda i: (0, 0)),
        pl.BlockSpec((3 * H, H), lambda i: (0, 0)),
        pl.BlockSpec((1, 3 * H), lambda i: (0, 0)),
    ],
    out_specs=[
        pl.BlockSpec((BLK, H), lambda i: (i, 0)),
        pl.BlockSpec((BLK, H), lambda i: (i, 0)),
    ],
    out_shape=[
        jax.ShapeDtypeStruct((N, H), jnp.float32),
        jax.ShapeDtypeStruct((N, H), jnp.float32),
    ],
)


@jax.jit
def kernel(h, c, edge_index, U_iou_W, U_f_W, U_f_b, b_iou):
    src = edge_index[0]
    dst = edge_index[1]
    src_p = jnp.concatenate(
        [src.reshape(NT, EPT_RAW),
         jnp.zeros((NT, PAD), jnp.int32)], axis=1).reshape(NT, NBLK, IB, K)
    dst_p = jnp.concatenate(
        [dst.reshape(NT, EPT_RAW),
         jnp.full((NT, PAD), N, jnp.int32)], axis=1).reshape(NT, NBLK, IB, K)
    h_sum, c_sum = _segment_sums(h, c, src_p, dst_p)
    h_new, c_new = _gates(h_sum, c_sum, U_f_W, U_f_b.reshape(1, H),
                          U_iou_W, b_iou)
    return (h_new, c_new)
